# baseline (speedup 1.0000x reference)
import functools

import jax
import jax.numpy as jnp
from jax import lax
from jax.experimental import pallas as pl
from jax.experimental.pallas import tpu as pltpu
from jax.experimental.pallas import tpu_sc as plsc

VOCAB = 1000000
EMB = 64
PAIR = 2 * EMB
T = 20
NB = 16384
B = NB * T
NC, NS = 2, 16
NW = NC * NS
B_PER_W = B // NW
NB_PER_W = NB // NW
CB = 8
CHUNK = CB * T
N_CHUNKS = B_PER_W // CHUNK
L = 16

_mesh = plsc.VectorSubcoreMesh(
    core_axis_name="c", subcore_axis_name="s", num_cores=NC, num_subcores=NS
)


@functools.partial(
    pl.kernel,
    out_type=jax.ShapeDtypeStruct((NB, T, EMB), jnp.float32),
    mesh=_mesh,
    scratch_types=[
        pltpu.VMEM((B_PER_W,), jnp.int32),
        pltpu.VMEM((B_PER_W,), jnp.int32),
        [pltpu.VMEM((CHUNK, PAIR), jnp.float32) for _ in range(2)],
        [pltpu.VMEM((CB, T, EMB), jnp.float32) for _ in range(2)],
        [pltpu.SemaphoreType.DMA for _ in range(2)],
        [pltpu.SemaphoreType.DMA for _ in range(2)],
    ],
)
def _gather(table_hbm, idx_hbm, out_hbm, idx_v, pid_v, pairs, stg, gsem, ssem):
    wid = lax.axis_index("s") * NC + lax.axis_index("c")
    base = wid * B_PER_W
    batch0 = wid * NB_PER_W
    pltpu.sync_copy(idx_hbm.at[pl.ds(base, B_PER_W)], idx_v)

    def pid_body(i, carry):
        r = idx_v[pl.ds(i * L, L)]
        pid_v[pl.ds(i * L, L)] = lax.shift_right_logical(r, 1)
        return carry

    lax.fori_loop(0, B_PER_W // L, pid_body, 0)

    def start_gather(c, b):
        return pltpu.async_copy(
            table_hbm.at[pid_v.at[pl.ds(c * CHUNK, CHUNK)]], pairs[b], gsem[b]
        )

    def wait_gather(c, b):
        pltpu.make_async_copy(
            table_hbm.at[pid_v.at[pl.ds(c * CHUNK, CHUNK)]], pairs[b], gsem[b]
        ).wait()

    def start_store(c, b):
        return pltpu.async_copy(
            stg[b], out_hbm.at[pl.ds(batch0 + c * CB, CB)], ssem[b]
        )

    def wait_store(c, b):
        pltpu.make_async_copy(
            stg[b], out_hbm.at[pl.ds(batch0 + c * CB, CB)], ssem[b]
        ).wait()

    lane = lax.iota(jnp.int32, L)

    def extract(c, b):
        def gbody(g, carry):
            r_vec = idx_v[pl.ds(c * CHUNK + g * L, L)]
            h_vec = lax.bitwise_and(r_vec, 1) * EMB
            i0 = g * L
            for l in range(L):
                off = h_vec[l]
                i = i0 + l
                bq = lax.div(i, T)
                tq = lax.rem(i, T)
                for j in range(EMB // L):
                    stg[b][bq, tq, pl.ds(j * L, L)] = pairs[b][
                        i, pl.ds(off + j * L, L)
                    ]
            return carry

        lax.fori_loop(0, CHUNK // L, gbody, 0)

    start_gather(0, 0)

    def loop_body(k, carry):
        c0 = 2 * k
        c1 = c0 + 1
        start_gather(c1, 1)
        wait_gather(c0, 0)

        @pl.when(k > 0)
        def _():
            wait_store(c0 - 2, 0)

        extract(c0, 0)
        start_store(c0, 0)

        @pl.when(k < N_CHUNKS // 2 - 1)
        def _():
            start_gather(c0 + 2, 0)

        wait_gather(c1, 1)

        @pl.when(k > 0)
        def _():
            wait_store(c1 - 2, 1)

        extract(c1, 1)
        start_store(c1, 1)
        return carry

    lax.fori_loop(0, N_CHUNKS // 2, loop_body, 0)
    wait_store(N_CHUNKS - 2, 0)
    wait_store(N_CHUNKS - 1, 1)


def kernel(token_ids, weight):
    wp = weight.reshape(VOCAB // 2, PAIR)
    return _gather(wp, token_ids.reshape(-1))


# speedup vs baseline: 1.4691x; 1.4691x over previous
import functools

import jax
import jax.numpy as jnp
from jax import lax
from jax.experimental import pallas as pl
from jax.experimental.pallas import tpu as pltpu
from jax.experimental.pallas import tpu_sc as plsc

VOCAB = 1000000
EMB = 64
T = 20
NB = 16384
B = NB * T
NC, NS = 2, 16
NW = NC * NS
B_PER_W = B // NW
NB_PER_W = NB // NW
CB = 16
CHUNK = CB * T
N_CHUNKS = B_PER_W // CHUNK
L = 16

_mesh = plsc.VectorSubcoreMesh(
    core_axis_name="c", subcore_axis_name="s", num_cores=NC, num_subcores=NS
)


@functools.partial(
    pl.kernel,
    out_type=jax.ShapeDtypeStruct((NB, T, EMB), jnp.float32),
    mesh=_mesh,
    scratch_types=[
        pltpu.VMEM((B_PER_W,), jnp.int32),
        [pltpu.VMEM((CB, T, EMB), jnp.float32) for _ in range(2)],
        [pltpu.SemaphoreType.DMA for _ in range(2)],
        [pltpu.SemaphoreType.DMA for _ in range(2)],
    ],
)
def _gather(table_hbm, idx_hbm, out_hbm, idx_v, rows, gsem, ssem):
    wid = lax.axis_index("s") * NC + lax.axis_index("c")
    base = wid * B_PER_W
    batch0 = wid * NB_PER_W
    pltpu.sync_copy(idx_hbm.at[pl.ds(base, B_PER_W)], idx_v)

    def issue_gathers(c, b):
        def gbody(g, carry):
            r_vec = idx_v[pl.ds(c * CHUNK + g * L, L)]
            i0 = g * L
            for l in range(L):
                r = r_vec[l]
                i = i0 + l
                bq = lax.div(i, T)
                tq = lax.rem(i, T)
                pltpu.async_copy(
                    table_hbm.at[pl.ds(r, 1)],
                    rows[b].at[bq, pl.ds(tq, 1)],
                    gsem[b],
                )
            return carry

        lax.fori_loop(0, CHUNK // L, gbody, 0)

    def drain_gathers(c, b):
        pltpu.make_async_copy(
            out_hbm.at[pl.ds(batch0 + c * CB, CB)], rows[b], gsem[b]
        ).wait()

    def start_store(c, b):
        return pltpu.async_copy(
            rows[b], out_hbm.at[pl.ds(batch0 + c * CB, CB)], ssem[b]
        )

    def wait_store(c, b):
        pltpu.make_async_copy(
            rows[b], out_hbm.at[pl.ds(batch0 + c * CB, CB)], ssem[b]
        ).wait()

    issue_gathers(0, 0)

    def loop_body(k, carry):
        c0 = 2 * k
        c1 = c0 + 1

        @pl.when(k > 0)
        def _():
            wait_store(c1 - 2, 1)

        issue_gathers(c1, 1)
        drain_gathers(c0, 0)
        start_store(c0, 0)

        @pl.when(k < N_CHUNKS // 2 - 1)
        def _():
            wait_store(c0, 0)
            issue_gathers(c0 + 2, 0)

        drain_gathers(c1, 1)
        start_store(c1, 1)
        return carry

    lax.fori_loop(0, N_CHUNKS // 2, loop_body, 0)
    wait_store(N_CHUNKS - 2, 0)
    wait_store(N_CHUNKS - 1, 1)


def kernel(token_ids, weight):
    return _gather(weight, token_ids.reshape(-1))


# speedup vs baseline: 1.4692x; 1.0001x over previous
import functools

import jax
import jax.numpy as jnp
from jax import lax
from jax.experimental import pallas as pl
from jax.experimental.pallas import tpu as pltpu
from jax.experimental.pallas import tpu_sc as plsc

VOCAB = 1000000
EMB = 64
T = 20
NB = 16384
B = NB * T
NC, NS = 2, 16
NW = NC * NS
B_PER_W = B // NW
NB_PER_W = NB // NW
CB = 16
CHUNK = CB * T
N_CHUNKS = B_PER_W // CHUNK
L = 16

_mesh = plsc.VectorSubcoreMesh(
    core_axis_name="c", subcore_axis_name="s", num_cores=NC, num_subcores=NS
)


@functools.partial(
    pl.kernel,
    out_type=jax.ShapeDtypeStruct((NB, T, EMB), jnp.float32),
    mesh=_mesh,
    scratch_types=[
        pltpu.VMEM((B_PER_W,), jnp.int32),
        [pltpu.VMEM((CHUNK, EMB), jnp.float32) for _ in range(2)],
        [pltpu.SemaphoreType.DMA for _ in range(2)],
        [pltpu.SemaphoreType.DMA for _ in range(2)],
    ],
)
def _gather(table_hbm, idx_hbm, out_hbm, idx_v, rows, gsem, ssem):
    wid = lax.axis_index("s") * NC + lax.axis_index("c")
    base = wid * B_PER_W
    batch0 = wid * NB_PER_W
    pltpu.sync_copy(idx_hbm.at[pl.ds(base, B_PER_W)], idx_v)

    def issue_gathers(c, b):
        def gbody(g, carry):
            r_vec = idx_v[pl.ds(c * CHUNK + g * L, L)]
            i0 = g * L
            for l in range(L):
                pltpu.async_copy(
                    table_hbm.at[pl.ds(r_vec[l], 1)],
                    rows[b].at[pl.ds(i0 + l, 1)],
                    gsem[b],
                )
            return carry

        lax.fori_loop(0, CHUNK // L, gbody, 0)

    def drain_gathers(b):
        pltpu.make_async_copy(
            table_hbm.at[pl.ds(0, CHUNK)], rows[b], gsem[b]
        ).wait()

    def start_stores(c, b):
        for q in range(CB):
            pltpu.async_copy(
                rows[b].at[pl.ds(q * T, T)],
                out_hbm.at[batch0 + c * CB + q],
                ssem[b],
            )

    def wait_stores(b):
        pltpu.make_async_copy(
            table_hbm.at[pl.ds(0, CHUNK)], rows[b], ssem[b]
        ).wait()

    issue_gathers(0, 0)

    def loop_body(k, carry):
        c0 = 2 * k
        c1 = c0 + 1

        @pl.when(k > 0)
        def _():
            wait_stores(1)

        issue_gathers(c1, 1)
        drain_gathers(0)
        start_stores(c0, 0)

        @pl.when(k < N_CHUNKS // 2 - 1)
        def _():
            wait_stores(0)
            issue_gathers(c0 + 2, 0)

        drain_gathers(1)
        start_stores(c1, 1)
        return carry

    lax.fori_loop(0, N_CHUNKS // 2, loop_body, 0)
    wait_stores(0)
    wait_stores(1)


def kernel(token_ids, weight):
    return _gather(weight, token_ids.reshape(-1))
